# baseline (speedup 1.0000x reference)
import jax
import jax.numpy as jnp
from jax import lax
from jax.experimental import pallas as pl
from jax.experimental.pallas import tpu as pltpu
from jax.experimental.pallas import tpu_sc as plsc

_B = 16384
_D = 64
_L = 16
_NC = 2
_NS = 16
_NW = _NC * _NS

_NB_SC = 4096
_B_TC = _B - _NB_SC
_SC_ROWS = _NB_SC // _NW


def _sc_body(sre_h, sim_h, rre_h, rim_h, ore_h, oim_h, out_h, *scratch):
    bufs = scratch[0:6]
    out_v = scratch[6]
    sem, osem = scratch[7], scratch[8]
    hbms = (sre_h, sim_h, rre_h, rim_h, ore_h, oim_h)

    wid = lax.axis_index("s") * _NC + lax.axis_index("c")
    base_col = _B_TC + wid * _SC_ROWS

    pend = [pltpu.async_copy(h.at[:, pl.ds(base_col, _SC_ROWS)], v, sem)
            for h, v in zip(hbms, bufs)]
    for h in pend:
        h.wait()
    sre_v, sim_v, rre_v, rim_v, ore_v, oim_v = bufs

    @plsc.parallel_loop(0, _SC_ROWS // _L, unroll=1)
    def grp_body(j):
        col = j * _L

        def d_body(d4, accs):
            new = []
            for q in range(4):
                d = d4 * 4 + q
                vs_re = sre_v[d, pl.ds(col, _L)]
                vs_im = sim_v[d, pl.ds(col, _L)]
                vr_re = rre_v[d, pl.ds(col, _L)]
                vr_im = rim_v[d, pl.ds(col, _L)]
                vo_re = ore_v[d, pl.ds(col, _L)]
                vo_im = oim_v[d, pl.ds(col, _L)]
                u = vs_re * vo_re + vs_im * vo_im
                w = vs_re * vo_im - vs_im * vo_re
                new.append(accs[q] + vr_re * u + vr_im * w)
            return tuple(new)

        accs = lax.fori_loop(
            0, _D // 4, d_body,
            tuple(jnp.zeros((_L,), jnp.float32) for _ in range(4)))
        out_v[pl.ds(col, _L)] = (accs[0] + accs[1]) + (accs[2] + accs[3])

    pltpu.async_copy(out_v, out_h.at[pl.ds(wid * _SC_ROWS, _SC_ROWS)],
                     osem).wait()


def _sc_score(sre, sim, rre, rim, ore, oim):
    mesh = plsc.VectorSubcoreMesh(core_axis_name="c", subcore_axis_name="s")
    f = pl.kernel(
        _sc_body,
        out_type=jax.ShapeDtypeStruct((_NB_SC,), jnp.float32),
        mesh=mesh,
        scratch_types=[pltpu.VMEM((_D, _SC_ROWS), jnp.float32)
                       for _ in range(6)]
                      + [pltpu.VMEM((_SC_ROWS,), jnp.float32)]
                      + [pltpu.SemaphoreType.DMA for _ in range(2)],
    )
    return f(sre, sim, rre, rim, ore, oim)


_TCC = 3072
_TCN = _B_TC // _TCC


def _tc_block(sre, sim, rre, rim, ore, oim, out):
    u = sre[...] * ore[...] + sim[...] * oim[...]
    w = sre[...] * oim[...] - sim[...] * ore[...]
    combo = rre[...] * u + rim[...] * w
    out[...] = jnp.sum(combo, axis=0, keepdims=True)


def _tc_score(sre, sim, rre, rim, ore, oim):
    in_spec = pl.BlockSpec((_D, _TCC), lambda i: (0, i))
    return pl.pallas_call(
        _tc_block,
        grid=(_TCN,),
        in_specs=[in_spec] * 6,
        out_specs=pl.BlockSpec((1, _TCC), lambda i: (0, i)),
        out_shape=jax.ShapeDtypeStruct((1, _B_TC), jnp.float32),
        compiler_params=pltpu.CompilerParams(
            dimension_semantics=("parallel",)),
    )(sre, sim, rre, rim, ore, oim)


@jax.jit
def _score(*cols):
    tc_out = _tc_score(*cols)
    sc_out = _sc_score(*cols)
    return jnp.concatenate([tc_out.reshape(-1), sc_out])


def kernel(s_re, s_im, r_re, r_im, o_re, o_im):
    cols = [jnp.squeeze(x, 1).T
            for x in (s_re, s_im, r_re, r_im, o_re, o_im)]
    return _score(*cols).reshape(_B, 1)


# speedup vs baseline: 2.8211x; 2.8211x over previous
import jax
import jax.numpy as jnp
from jax.experimental import pallas as pl
from jax.experimental.pallas import tpu as pltpu

_B = 16384
_D = 64
_TCC = 4096
_TCN = _B // _TCC


def _tc_block(sre, sim, rre, rim, ore, oim, out):
    u = sre[...] * ore[...] + sim[...] * oim[...]
    w = sre[...] * oim[...] - sim[...] * ore[...]
    combo = rre[...] * u + rim[...] * w
    out[...] = jnp.sum(combo, axis=0, keepdims=True)


@jax.jit
def _tc_score(sre, sim, rre, rim, ore, oim):
    in_spec = pl.BlockSpec((_D, _TCC), lambda i: (0, i))
    return pl.pallas_call(
        _tc_block,
        grid=(_TCN,),
        in_specs=[in_spec] * 6,
        out_specs=pl.BlockSpec((1, _TCC), lambda i: (0, i)),
        out_shape=jax.ShapeDtypeStruct((1, _B), jnp.float32),
        compiler_params=pltpu.CompilerParams(
            dimension_semantics=("arbitrary",)),
    )(sre, sim, rre, rim, ore, oim)


def kernel(s_re, s_im, r_re, r_im, o_re, o_im):
    cols = [jnp.squeeze(x, 1).T
            for x in (s_re, s_im, r_re, r_im, o_re, o_im)]
    return _tc_score(*cols).T
